# baseline (speedup 1.0000x reference)
import functools

import jax
import jax.numpy as jnp
from jax import lax
from jax.experimental import pallas as pl
from jax.experimental.pallas import tpu as pltpu
from jax.experimental.pallas import tpu_sc as plsc

N = 10000
E = 160000
D = 128
DE = 16
MEG = 256
BGRAPH = 16
GROUPS = 2
PASSES = 2

NC, NS = 2, 16
N_PAD = 10240
E_PAD = 163840
BATCH = 128
EDGES_PER_W = E_PAD // (NC * NS)
NBATCH = EDGES_PER_W // BATCH
ROWS_PER_TILE = N_PAD // NS
CW = 128
NCH = 4
NHALF = N_PAD // NC
ETILE = E_PAD // NS
NBATCH_E = ETILE // BATCH
ACC_ROWS = 5248
ZROWS = ACC_ROWS // NS
CROWS = NHALF // NS
RB = 640
NRB = N_PAD // RB
ERB = E_PAD // RB

_NEG = -1.0e30


def _enc_body(x_ref, w_ref, b_ref, o_ref):
    o_ref[...] = (
        jnp.dot(x_ref[...], w_ref[...], preferred_element_type=jnp.float32)
        + b_ref[...]
    )


def _enc(x, w, b):
    return pl.pallas_call(
        _enc_body,
        grid=(NRB,),
        in_specs=[
            pl.BlockSpec((RB, D), lambda i: (i, 0)),
            pl.BlockSpec((D, D), lambda i: (0, 0)),
            pl.BlockSpec((1, D), lambda i: (0, 0)),
        ],
        out_specs=pl.BlockSpec((RB, D), lambda i: (i, 0)),
        out_shape=jax.ShapeDtypeStruct((N_PAD, D), jnp.float32),
    )(x, w, b)


def _prep_body(h_ref, wd_ref, ws_ref, *o_refs):
    hb = h_ref[...]
    for c in range(4):
        o_refs[c][...] = jnp.dot(
            hb, wd_ref[:, c * 128:(c + 1) * 128],
            preferred_element_type=jnp.float32)
        o_refs[4 + c][...] = jnp.dot(
            hb, ws_ref[:, c * 128:(c + 1) * 128],
            preferred_element_type=jnp.float32)


def _prep(h, wd, ws):
    blk = pl.BlockSpec((RB, 128), lambda i: (i, 0))
    return pl.pallas_call(
        _prep_body,
        grid=(NRB,),
        in_specs=[
            pl.BlockSpec((RB, D), lambda i: (i, 0)),
            pl.BlockSpec((D, 512), lambda i: (0, 0)),
            pl.BlockSpec((D, 512), lambda i: (0, 0)),
        ],
        out_specs=[blk] * 8,
        out_shape=[jax.ShapeDtypeStruct((N_PAD, 128), jnp.float32)] * 8,
    )(h, wd, ws)


def _ecc_body(ef_ref, we_ref, be_ref, w1c_ref, b1_ref, *o_refs):
    e = (jnp.dot(ef_ref[...], we_ref[...], preferred_element_type=jnp.float32)
         + be_ref[...])
    rows = (pl.program_id(0) * RB
            + lax.broadcasted_iota(jnp.int32, (RB, 1), 0))
    keep = rows < E
    for j in range(8):
        v = (jnp.dot(e, w1c_ref[j], preferred_element_type=jnp.float32)
             + b1_ref[j])
        o_refs[j][...] = jnp.where(keep, v, _NEG)


def _ecc(efp, we, be, w1c_all, b1_all):
    blk = pl.BlockSpec((RB, CW), lambda i: (i, 0))
    return pl.pallas_call(
        _ecc_body,
        grid=(ERB,),
        in_specs=[
            pl.BlockSpec((RB, DE), lambda i: (i, 0)),
            pl.BlockSpec((DE, DE), lambda i: (0, 0)),
            pl.BlockSpec((1, DE), lambda i: (0, 0)),
            pl.BlockSpec((8, DE, 128), lambda i: (0, 0, 0)),
            pl.BlockSpec((8, 1, 128), lambda i: (0, 0, 0)),
        ],
        out_specs=[blk] * 8,
        out_shape=[jax.ShapeDtypeStruct((E_PAD, CW), jnp.float32)] * 8,
    )(efp, we, be, w1c_all, b1_all)


def _post_body(p_ref, h_ref, skip_ref, deg_ref, w2f_ref, w2r_ref,
               b2f_ref, b2r_ref, wih_ref, bih_ref, whh_ref, bhh_ref,
               hnew_ref, hskip_ref):
    p = p_ref[...]
    sf = jnp.concatenate([p[0, 0], p[0, 1]], axis=1)
    sr = jnp.concatenate([p[0, 2], p[0, 3]], axis=1)
    dg = deg_ref[...]
    degd = dg[0, 0, :, 0:1] + dg[1, 0, :, 0:1]
    degs = dg[0, 1, :, 0:1] + dg[1, 1, :, 0:1]
    m = (jnp.dot(sf, w2f_ref[...], preferred_element_type=jnp.float32)
         + jnp.dot(sr, w2r_ref[...], preferred_element_type=jnp.float32)
         + degd * b2f_ref[...] + degs * b2r_ref[...])
    hb = h_ref[...]
    gi = jnp.dot(m, wih_ref[...], preferred_element_type=jnp.float32) + bih_ref[...]
    gh = jnp.dot(hb, whh_ref[...], preferred_element_type=jnp.float32) + bhh_ref[...]
    r = jax.nn.sigmoid(gi[:, 0:D] + gh[:, 0:D])
    z = jax.nn.sigmoid(gi[:, D:2 * D] + gh[:, D:2 * D])
    n = jnp.tanh(gi[:, 2 * D:3 * D] + r * gh[:, 2 * D:3 * D])
    hn = (1.0 - z) * n + z * hb
    hnew_ref[...] = hn
    hskip_ref[...] = hn + skip_ref[...]


def _post(p, h, skip, degp, w2f, w2r, b2f, b2r, wih, bih, whh, bhh):
    nblk = pl.BlockSpec((RB, D), lambda i: (i, 0))
    return pl.pallas_call(
        _post_body,
        grid=(NRB,),
        in_specs=[
            pl.BlockSpec((1, NCH, RB, CW),
                         lambda i: (i // (NHALF // RB), 0, i % (NHALF // RB), 0)),
            nblk,
            nblk,
            pl.BlockSpec((NC, 2, RB, 16), lambda i: (0, 0, i, 0)),
            pl.BlockSpec((MEG, MEG), lambda i: (0, 0)),
            pl.BlockSpec((MEG, MEG), lambda i: (0, 0)),
            pl.BlockSpec((1, MEG), lambda i: (0, 0)),
            pl.BlockSpec((1, MEG), lambda i: (0, 0)),
            pl.BlockSpec((MEG, 3 * D), lambda i: (0, 0)),
            pl.BlockSpec((1, 3 * D), lambda i: (0, 0)),
            pl.BlockSpec((D, 3 * D), lambda i: (0, 0)),
            pl.BlockSpec((1, 3 * D), lambda i: (0, 0)),
        ],
        out_specs=[nblk, nblk],
        out_shape=[jax.ShapeDtypeStruct((N_PAD, D), jnp.float32)] * 2,
    )(p, h, skip, degp, w2f, w2r, b2f, b2r, wih, bih, whh, bhh)


def _pool_body(g0_ref, g1_ref, gidx_ref, o_ref):
    @pl.when(pl.program_id(0) == 0)
    def _():
        o_ref[...] = jnp.zeros_like(o_ref)

    gid = gidx_ref[0, 0, :]
    oh = (gid[None, :]
          == lax.broadcasted_iota(jnp.int32, (BGRAPH, RB), 0)).astype(jnp.float32)
    xb = jnp.concatenate([g0_ref[...], g1_ref[...]], axis=1)
    o_ref[...] += jnp.dot(oh, xb, preferred_element_type=jnp.float32)


def _pool(g0, g1, gidx3):
    nblk = pl.BlockSpec((RB, D), lambda i: (i, 0))
    return pl.pallas_call(
        _pool_body,
        grid=(NRB,),
        in_specs=[
            nblk,
            nblk,
            pl.BlockSpec((1, 1, RB), lambda i: (i, 0, 0)),
        ],
        out_specs=pl.BlockSpec((BGRAPH, 2 * D), lambda i: (0, 0)),
        out_shape=jax.ShapeDtypeStruct((BGRAPH, 2 * D), jnp.float32),
    )(g0, g1, gidx3)


_SC_MESH = plsc.VectorSubcoreMesh(core_axis_name="c", subcore_axis_name="s")


@functools.partial(
    pl.kernel,
    out_type=jax.ShapeDtypeStruct((NC, NCH, NHALF, CW), jnp.float32),
    mesh=_SC_MESH,
    scratch_types=[
        pltpu.VMEM_SHARED((ACC_ROWS, CW), jnp.float32),
        pltpu.VMEM((ETILE,), jnp.int32),
        pltpu.VMEM((ETILE,), jnp.int32),
        pltpu.VMEM((8, CW), jnp.float32),
        pltpu.VMEM((BATCH, CW), jnp.float32),
        pltpu.VMEM((BATCH, CW), jnp.float32),
        pltpu.VMEM((BATCH, CW), jnp.float32),
        pltpu.VMEM((BATCH, CW), jnp.float32),
        pltpu.VMEM((BATCH,), jnp.int32),
    ],
)
def _edge_kernel(td0, td1, td2, td3, ts0, ts1, ts2, ts3,
                 ec0, ec1, ec2, ec3,
                 dsti, srci, out,
                 acc, idxd_all, idxs_all, zbuf, rowsd, rowss, ecv, vals,
                 sidx):
    cid = lax.axis_index("c")
    sid = lax.axis_index("s")
    ebase = sid * ETILE
    lo = cid * NHALF

    pltpu.sync_copy(dsti.at[pl.ds(ebase, ETILE)], idxd_all)
    pltpu.sync_copy(srci.at[pl.ds(ebase, ETILE)], idxs_all)

    def _zb(r, cc):
        for k in range(CW // 16):
            zbuf[r, pl.ds(k * 16, 16)] = jnp.zeros((16,), jnp.float32)
        return cc
    lax.fori_loop(0, 8, _zb, 0)

    tds = [td0, td1, td2, td3]
    tss = [ts0, ts1, ts2, ts3]
    ecs = [ec0, ec1, ec2, ec3]
    for c in range(NCH):
        zb0 = sid * ZROWS

        def _zero(z, cc):
            pltpu.sync_copy(zbuf, acc.at[pl.ds(zb0 + z * 8, 8), :])
            return cc
        lax.fori_loop(0, ZROWS // 8, _zero, 0)
        plsc.subcore_barrier()

        def _batch(b, cc, c=c):
            lbase = b * BATCH
            idx = idxd_all if c < 2 else idxs_all

            def _sx(k, kk):
                s16 = pl.ds(k * 16, 16)
                v = idx[pl.ds(lbase + k * 16, 16)]
                vloc = v - lo
                ok = (v >= lo) & (vloc < NHALF)
                sidx[s16] = jnp.where(ok, vloc, NHALF)
                return kk
            lax.fori_loop(0, BATCH // 16, _sx, 0)

            pltpu.sync_copy(tds[c].at[idxd_all.at[pl.ds(lbase, BATCH)]],
                            rowsd)
            pltpu.sync_copy(tss[c].at[idxs_all.at[pl.ds(lbase, BATCH)]],
                            rowss)
            pltpu.sync_copy(ecs[c].at[pl.ds(ebase + lbase, BATCH), :], ecv)

            def _row(r, rr):
                for k in range(CW // 16):
                    s = pl.ds(k * 16, 16)
                    vals[r, s] = jnp.maximum(
                        rowsd[r, s] + rowss[r, s] + ecv[r, s], 0.0)
                return rr
            lax.fori_loop(0, BATCH, _row, 0)
            pltpu.sync_copy(vals, acc.at[sidx], add=True)
            return cc
        lax.fori_loop(0, NBATCH_E, _batch, 0)
        plsc.subcore_barrier()
        pltpu.sync_copy(acc.at[pl.ds(sid * CROWS, CROWS), :],
                        out.at[cid, c, pl.ds(sid * CROWS, CROWS), :])


@functools.partial(
    pl.kernel,
    out_type=jax.ShapeDtypeStruct((NC, 2, N_PAD, 16), jnp.float32),
    mesh=_SC_MESH,
    scratch_types=[
        pltpu.VMEM_SHARED((N_PAD, 16), jnp.float32),
        pltpu.VMEM_SHARED((N_PAD, 16), jnp.float32),
        pltpu.VMEM((BATCH, 16), jnp.float32),
        pltpu.VMEM((BATCH, 16), jnp.float32),
        pltpu.VMEM((BATCH,), jnp.int32),
        pltpu.VMEM((BATCH,), jnp.int32),
    ],
)
def _deg_kernel(dsti, srci, valid, out, accd, accs_, vbuf, zbuf, idxd, idxs):
    cid = lax.axis_index("c")
    sid = lax.axis_index("s")
    wid = cid * NS + sid
    ebase = wid * EDGES_PER_W
    rbase = sid * ROWS_PER_TILE

    def _zb(r, cc):
        zbuf[r, :] = jnp.zeros((16,), jnp.float32)
        return cc
    lax.fori_loop(0, BATCH, _zb, 0)

    for z in range(ROWS_PER_TILE // BATCH):
        pltpu.sync_copy(zbuf, accd.at[pl.ds(rbase + z * BATCH, BATCH), :])
        pltpu.sync_copy(zbuf, accs_.at[pl.ds(rbase + z * BATCH, BATCH), :])
    plsc.subcore_barrier()

    def _batch(b, cc):
        base = ebase + b * BATCH
        pltpu.sync_copy(dsti.at[pl.ds(base, BATCH)], idxd)
        pltpu.sync_copy(srci.at[pl.ds(base, BATCH)], idxs)
        pltpu.sync_copy(valid.at[pl.ds(base, BATCH), :], vbuf)
        pltpu.sync_copy(vbuf, accd.at[idxd], add=True)
        pltpu.sync_copy(vbuf, accs_.at[idxs], add=True)
        return cc
    lax.fori_loop(0, NBATCH, _batch, 0)
    plsc.subcore_barrier()
    pltpu.sync_copy(accd.at[pl.ds(rbase, ROWS_PER_TILE), :],
                    out.at[cid, 0, pl.ds(rbase, ROWS_PER_TILE), :])
    pltpu.sync_copy(accs_.at[pl.ds(rbase, ROWS_PER_TILE), :],
                    out.at[cid, 1, pl.ds(rbase, ROWS_PER_TILE), :])


def kernel(x, edge_feat, params, edge_index, graph_idx, batch_size):
    del batch_size
    xp = jnp.pad(x, ((0, N_PAD - N), (0, 0)))
    dst = jnp.pad(edge_index[1], (0, E_PAD - E)).astype(jnp.int32)
    src = jnp.pad(edge_index[0], (0, E_PAD - E)).astype(jnp.int32)
    efp = jnp.pad(edge_feat, ((0, E_PAD - E), (0, 0)))
    gidx3 = jnp.pad(graph_idx.astype(jnp.int32), (0, N_PAD - N),
                    constant_values=BGRAPH).reshape(NRB, 1, RB)
    valid = ((jnp.arange(E_PAD) < E).astype(jnp.float32)[:, None]
             * jnp.ones((1, 16), jnp.float32))

    gp = params["groups"]
    wds, wss, w1cs, b1s = [], [], [], []
    for g in range(GROUPS):
        w1f = gp[g]["msg"]["W1"]
        w1r = gp[g]["rev"]["W1"]
        b1f = gp[g]["msg"]["b1"]
        b1r = gp[g]["rev"]["b1"]
        wds.append(jnp.concatenate([w1f[0:D], w1r[D:2 * D]], axis=1))
        wss.append(jnp.concatenate([w1f[D:2 * D], w1r[0:D]], axis=1))
        w1cs.append(jnp.stack([
            w1f[2 * D:, 0:128], w1f[2 * D:, 128:256],
            w1r[2 * D:, 0:128], w1r[2 * D:, 128:256]]))
        b1s.append(jnp.stack([
            b1f[0:128].reshape(1, 128), b1f[128:256].reshape(1, 128),
            b1r[0:128].reshape(1, 128), b1r[128:256].reshape(1, 128)]))
    w1c_all = jnp.concatenate(w1cs)
    b1_all = jnp.concatenate(b1s)

    h = _enc(xp, params["W_enc_x"], params["b_enc_x"].reshape(1, D))
    eccs = _ecc(efp, params["W_enc_e"], params["b_enc_e"].reshape(1, DE),
                w1c_all, b1_all)
    degp = _deg_kernel(dst, src, valid)

    feats = []
    for g in range(GROUPS):
        skip = h
        w2f = gp[g]["msg"]["W2"]
        w2r = gp[g]["rev"]["W2"]
        b2f = gp[g]["msg"]["b2"].reshape(1, MEG)
        b2r = gp[g]["rev"]["b2"].reshape(1, MEG)
        wih = gp[g]["gru"]["Wih"]
        bih = gp[g]["gru"]["bih"].reshape(1, 3 * D)
        whh = gp[g]["gru"]["Whh"]
        bhh = gp[g]["gru"]["bhh"].reshape(1, 3 * D)
        for _ in range(PASSES):
            t = _prep(h, wds[g], wss[g])
            ecg = eccs[4 * g:4 * g + 4]
            p = _edge_kernel(t[0], t[1], t[2], t[3],
                             t[4], t[5], t[6], t[7],
                             ecg[0], ecg[1], ecg[2], ecg[3],
                             dst, src)
            h, hs = _post(p, h, skip, degp, w2f, w2r, b2f, b2r,
                          wih, bih, whh, bhh)
        feats.append(h)
        h = hs

    return _pool(feats[0], feats[1], gidx3)
